# baseline (speedup 1.0000x reference)
import functools

import jax
import jax.numpy as jnp
from jax.experimental import pallas as pl
from jax.experimental.pallas import tpu as pltpu

TN = 256

_INV_SQRT2 = 0.7071067811865476


def _gelu_exact(v):
    return 0.5 * v * (1.0 + jax.lax.erf(v * _INV_SQRT2))


def _body(nt, t_ref, x_ref, w1a_ref, b1a_ref, w1b_ref, b1b_ref,
          w2a_ref, b2a_ref, w2b_ref, b2b_ref, tt_ref, pos_ref,
          gamma_ref, beta_ref, o_ref, gbuf):
    f32 = jnp.float32
    bf16 = jnp.bfloat16
    d1 = w1a_ref.shape[0]
    d2 = w2a_ref.shape[0]
    g = pl.program_id(0)
    wslot = g % 2
    rslot = (g + 1) % 2

    h1 = jnp.dot(gbuf[rslot, 0], w1b_ref[...], preferred_element_type=f32) + b1b_ref[...]
    h2 = jnp.dot(gbuf[rslot, 1], w2b_ref[...], preferred_element_type=f32) + b2b_ref[...]
    m1 = t_ref[...] == 0
    h = jnp.where(m1, h1, h2)
    h = h + jnp.where(m1, tt_ref[0:1, :], tt_ref[1:2, :]) + pos_ref[...]
    mu = jnp.mean(h, axis=-1, keepdims=True)
    c = h - mu
    var = jnp.mean(c * c, axis=-1, keepdims=True)
    o_ref[...] = c * jax.lax.rsqrt(var + 1e-5) * gamma_ref[...] + beta_ref[...]

    x = x_ref[...].astype(bf16)
    a1 = jnp.dot(x[:, :d1], w1a_ref[...], preferred_element_type=f32)
    gbuf[wslot, 0] = _gelu_exact(a1 + b1a_ref[...]).astype(bf16)
    a2 = jnp.dot(x[:, :d2], w2a_ref[...], preferred_element_type=f32)
    gbuf[wslot, 1] = _gelu_exact(a2 + b2a_ref[...]).astype(bf16)


def kernel(x, token_type_ids, W1a, b1a, W1b, b1b, W2a, b2a, W2b, b2b,
           type_table, pos_table, gamma, beta):
    B, L, Dx = x.shape
    DM = W1a.shape[1]
    N = B * L
    nt = N // TN
    pos_blocks = L // TN

    xf = x.reshape(N, Dx)
    tcol = token_type_ids.reshape(N, 1)

    const = lambda g: (0, 0)
    out = pl.pallas_call(
        functools.partial(_body, nt),
        grid=(nt + 1,),
        in_specs=[
            pl.BlockSpec((TN, 1), lambda g: (jnp.maximum(g - 1, 0), 0)),
            pl.BlockSpec((TN, Dx), lambda g: (jnp.minimum(g, nt - 1), 0)),
            pl.BlockSpec(W1a.shape, const),
            pl.BlockSpec((1, DM), const),
            pl.BlockSpec(W1b.shape, const),
            pl.BlockSpec((1, DM), const),
            pl.BlockSpec(W2a.shape, const),
            pl.BlockSpec((1, DM), const),
            pl.BlockSpec(W2b.shape, const),
            pl.BlockSpec((1, DM), const),
            pl.BlockSpec((2, DM), const),
            pl.BlockSpec((TN, DM),
                         lambda g: (jnp.maximum(g - 1, 0) % pos_blocks, 0)),
            pl.BlockSpec((1, DM), const),
            pl.BlockSpec((1, DM), const),
        ],
        out_specs=pl.BlockSpec((TN, DM), lambda g: (jnp.maximum(g - 1, 0), 0)),
        out_shape=jax.ShapeDtypeStruct((N, DM), jnp.float32),
        scratch_shapes=[
            pltpu.VMEM((2, 2, TN, DM), jnp.bfloat16),
        ],
        compiler_params=pltpu.CompilerParams(
            dimension_semantics=("arbitrary",),
        ),
    )(tcol, xf, W1a.astype(jnp.bfloat16), b1a.reshape(1, DM), W1b,
      b1b.reshape(1, DM), W2a.astype(jnp.bfloat16), b2a.reshape(1, DM), W2b,
      b2b.reshape(1, DM), type_table, pos_table, gamma.reshape(1, DM),
      beta.reshape(1, DM))

    return out.reshape(B, L, DM)


# speedup vs baseline: 1.0609x; 1.0609x over previous
import jax
import jax.numpy as jnp
from jax.experimental import pallas as pl
from jax.experimental.pallas import tpu as pltpu

TN = 256

_INV_SQRT2 = 0.7071067811865476


def _gelu_exact(v):
    return 0.5 * v * (1.0 + jax.lax.erf(v * _INV_SQRT2))


def _body(t_ref, x_ref, w1a_hbm, w1b_hbm, w2a_hbm, w2b_hbm, p_ref, pos_ref,
          o_ref, w1a_v, w2a_v, w1b_v, w2b_v, s1a, s2a, s1b, s2b):
    f32 = jnp.float32
    d1 = w1a_hbm.shape[0]
    d2 = w2a_hbm.shape[0]
    g = pl.program_id(0)
    p = p_ref[...]

    @pl.when(g == 0)
    def _start_and_wait_l1():
        pltpu.make_async_copy(w1a_hbm, w1a_v, s1a).start()
        pltpu.make_async_copy(w2a_hbm, w2a_v, s2a).start()
        pltpu.make_async_copy(w1b_hbm, w1b_v, s1b).start()
        pltpu.make_async_copy(w2b_hbm, w2b_v, s2b).start()
        pltpu.make_async_copy(w1a_hbm, w1a_v, s1a).wait()
        pltpu.make_async_copy(w2a_hbm, w2a_v, s2a).wait()

    x = x_ref[...]
    g1 = _gelu_exact(
        jnp.dot(x[:, :d1], w1a_v[...], preferred_element_type=f32)
        + p[0:1, :]).astype(jnp.bfloat16)
    g2 = _gelu_exact(
        jnp.dot(x[:, :d2], w2a_v[...], preferred_element_type=f32)
        + p[2:3, :]).astype(jnp.bfloat16)

    @pl.when(g == 0)
    def _wait_l2():
        pltpu.make_async_copy(w1b_hbm, w1b_v, s1b).wait()
        pltpu.make_async_copy(w2b_hbm, w2b_v, s2b).wait()

    h1 = jnp.dot(g1, w1b_v[...], preferred_element_type=f32) + p[1:2, :]
    h2 = jnp.dot(g2, w2b_v[...], preferred_element_type=f32) + p[3:4, :]

    m1 = t_ref[...] == 0
    h = jnp.where(m1, h1, h2)
    h = h + jnp.where(m1, p[6:7, :], p[7:8, :]) + pos_ref[...]

    mu = jnp.mean(h, axis=-1, keepdims=True)
    c = h - mu
    var = jnp.mean(c * c, axis=-1, keepdims=True)
    o_ref[...] = c * jax.lax.rsqrt(var + 1e-5) * p[4:5, :] + p[5:6, :]


def kernel(x, token_type_ids, W1a, b1a, W1b, b1b, W2a, b2a, W2b, b2b,
           type_table, pos_table, gamma, beta):
    B, L, Dx = x.shape
    DM = W1a.shape[1]
    D1 = W1a.shape[0]
    D2 = W2a.shape[0]
    N = B * L
    n_tiles = N // TN
    pos_blocks = L // TN

    xf = x.reshape(N, Dx)
    tcol = token_type_ids.reshape(N, 1)
    params = jnp.concatenate(
        [b1a[None], b1b[None], b2a[None], b2b[None],
         gamma[None], beta[None], type_table], axis=0)

    const = lambda g: (0, 0)
    hbm = pl.BlockSpec(memory_space=pl.ANY)
    out = pl.pallas_call(
        _body,
        grid=(n_tiles,),
        in_specs=[
            pl.BlockSpec((TN, 1), lambda g: (g, 0)),
            pl.BlockSpec((TN, Dx), lambda g: (g, 0)),
            hbm,
            hbm,
            hbm,
            hbm,
            pl.BlockSpec((8, DM), const),
            pl.BlockSpec((TN, DM), lambda g: (g % pos_blocks, 0)),
        ],
        out_specs=pl.BlockSpec((TN, DM), lambda g: (g, 0)),
        out_shape=jax.ShapeDtypeStruct((N, DM), jnp.float32),
        scratch_shapes=[
            pltpu.VMEM((D1, DM), jnp.float32),
            pltpu.VMEM((D2, DM), jnp.float32),
            pltpu.VMEM((DM, DM), jnp.float32),
            pltpu.VMEM((DM, DM), jnp.float32),
            pltpu.SemaphoreType.DMA,
            pltpu.SemaphoreType.DMA,
            pltpu.SemaphoreType.DMA,
            pltpu.SemaphoreType.DMA,
        ],
        compiler_params=pltpu.CompilerParams(
            dimension_semantics=("arbitrary",),
        ),
    )(tcol, xf, W1a, W1b, W2a, W2b, params, pos_table)

    return out.reshape(B, L, DM)


# speedup vs baseline: 1.1509x; 1.0849x over previous
import jax
import jax.numpy as jnp
from jax.experimental import pallas as pl
from jax.experimental.pallas import tpu as pltpu

TN = 256

_INV_SQRT2 = 0.7071067811865476


def _gelu_exact(v):
    return 0.5 * v * (1.0 + jax.lax.erf(v * _INV_SQRT2))


def _fused_body(t_ref, x_ref, w1a_ref, b1a_ref, w1b_ref, b1b_ref,
                w2a_ref, b2a_ref, w2b_ref, b2b_ref, tt_ref, pos_ref,
                gamma_ref, beta_ref, o_ref):
    f32 = jnp.float32
    d1 = w1a_ref.shape[0]
    d2 = w2a_ref.shape[0]
    x = x_ref[...]

    g1 = _gelu_exact(
        jnp.dot(x[:, :d1], w1a_ref[...], preferred_element_type=f32) + b1a_ref[...])
    h1 = jnp.dot(g1, w1b_ref[...], preferred_element_type=f32) + b1b_ref[...]

    g2 = _gelu_exact(
        jnp.dot(x[:, :d2], w2a_ref[...], preferred_element_type=f32) + b2a_ref[...])
    h2 = jnp.dot(g2, w2b_ref[...], preferred_element_type=f32) + b2b_ref[...]

    m1 = t_ref[...] == 0
    h = jnp.where(m1, h1, h2)
    h = h + jnp.where(m1, tt_ref[0:1, :], tt_ref[1:2, :]) + pos_ref[...]

    mu = jnp.mean(h, axis=-1, keepdims=True)
    c = h - mu
    var = jnp.mean(c * c, axis=-1, keepdims=True)
    o_ref[...] = c * jax.lax.rsqrt(var + 1e-5) * gamma_ref[...] + beta_ref[...]


def kernel(x, token_type_ids, W1a, b1a, W1b, b1b, W2a, b2a, W2b, b2b,
           type_table, pos_table, gamma, beta):
    B, L, Dx = x.shape
    DM = W1a.shape[1]
    N = B * L
    n_tiles = N // TN
    pos_blocks = L // TN

    xf = x.reshape(N, Dx)
    tcol = token_type_ids.reshape(N, 1)

    const = lambda p, j: (0, 0)
    tile = lambda p, j: (j * pos_blocks + p, 0)
    out = pl.pallas_call(
        _fused_body,
        grid=(pos_blocks, n_tiles // pos_blocks),
        in_specs=[
            pl.BlockSpec((TN, 1), tile),
            pl.BlockSpec((TN, Dx), tile),
            pl.BlockSpec(W1a.shape, const),
            pl.BlockSpec((1, DM), const),
            pl.BlockSpec(W1b.shape, const),
            pl.BlockSpec((1, DM), const),
            pl.BlockSpec(W2a.shape, const),
            pl.BlockSpec((1, DM), const),
            pl.BlockSpec(W2b.shape, const),
            pl.BlockSpec((1, DM), const),
            pl.BlockSpec((2, DM), const),
            pl.BlockSpec((TN, DM), lambda p, j: (p, 0)),
            pl.BlockSpec((1, DM), const),
            pl.BlockSpec((1, DM), const),
        ],
        out_specs=pl.BlockSpec((TN, DM), tile),
        out_shape=jax.ShapeDtypeStruct((N, DM), jnp.float32),
        compiler_params=pltpu.CompilerParams(
            dimension_semantics=("arbitrary", "arbitrary"),
        ),
    )(tcol, xf, W1a, b1a.reshape(1, DM), W1b, b1b.reshape(1, DM),
      W2a, b2a.reshape(1, DM), W2b, b2b.reshape(1, DM),
      type_table, pos_table, gamma.reshape(1, DM), beta.reshape(1, DM))

    return out.reshape(B, L, DM)
